# baseline (speedup 1.0000x reference)
import functools

import jax
import jax.numpy as jnp
from jax import lax
from jax.experimental import pallas as pl
from jax.experimental.pallas import tpu as pltpu
from jax.experimental.pallas import tpu_sc as plsc

N = 10000
E = 320000
D_IN = 128
D_H = 128
D_OUT = 64

NC = 2
NS = 16
NW = NC * NS
EPW = E // NW
CH = 80
NCHUNK = EPW // CH
RPS = N // NS
DEGW = 128

@functools.cache
def _mesh():
    return plsc.VectorSubcoreMesh(
        core_axis_name="c", subcore_axis_name="s",
        num_cores=NC, num_subcores=NS)


def _sc_degree(dst):
    ones_rows = jnp.ones((CH, DEGW), jnp.float32)
    zero_rows = jnp.zeros((RPS, DEGW), jnp.float32)

    @functools.partial(
        pl.kernel,
        out_type=jax.ShapeDtypeStruct((NC, NS, RPS, DEGW), jnp.float32),
        mesh=_mesh(),
        scratch_types=[
            pltpu.VMEM((CH,), jnp.int32),
            pltpu.VMEM((CH, DEGW), jnp.float32),
            pltpu.VMEM_SHARED((N, DEGW), jnp.float32),
        ],
    )
    def k(dst_hbm, ones_hbm, zero_hbm, out_hbm, dstv, onesv, acc):
        c = lax.axis_index("c")
        s = lax.axis_index("s")
        pltpu.sync_copy(zero_hbm, acc.at[pl.ds(s * RPS, RPS)])
        pltpu.sync_copy(ones_hbm, onesv)
        plsc.subcore_barrier()
        base_w = (c * NS + s) * EPW

        @pl.loop(0, NCHUNK)
        def _(ci):
            pltpu.sync_copy(dst_hbm.at[pl.ds(base_w + ci * CH, CH)], dstv)
            pltpu.sync_copy(onesv, acc.at[dstv], add=True)

        plsc.subcore_barrier()
        pltpu.sync_copy(acc.at[pl.ds(s * RPS, RPS)], out_hbm.at[c, s])

    return k(dst, ones_rows, zero_rows).reshape(NC, N, DEGW)


ACC_N = 10240
EPW_P = ACC_N
E_P = NW * EPW_P
CHP = 128
NCHUNK_P = EPW_P // CHP
RPS_P = ACC_N // NS


def _sc_aggregate(src_pad, dst_pad, y, d):
    zero_rows = jnp.zeros((RPS_P, d), jnp.float32)

    @functools.partial(
        pl.kernel,
        out_type=jax.ShapeDtypeStruct((NC, NS, RPS_P, d), jnp.float32),
        mesh=_mesh(),
        scratch_types=[
            pltpu.VMEM((CHP,), jnp.int32),
            pltpu.VMEM((CHP,), jnp.int32),
            pltpu.VMEM((CHP, d), jnp.float32),
            pltpu.VMEM((CHP,), jnp.int32),
            pltpu.VMEM((CHP,), jnp.int32),
            pltpu.VMEM((CHP, d), jnp.float32),
            pltpu.VMEM_SHARED((ACC_N, d), jnp.float32),
            pltpu.SemaphoreType.DMA,
            pltpu.SemaphoreType.DMA,
        ],
    )
    def k(src_hbm, dst_hbm, y_hbm, zero_hbm, out_hbm,
          sa, da, ra, sb, db, rb, acc, gsa, gsb):
        c = lax.axis_index("c")
        s = lax.axis_index("s")
        pltpu.sync_copy(zero_hbm, acc.at[pl.ds(s * RPS_P, RPS_P)])
        plsc.subcore_barrier()
        base_w = (c * NS + s) * EPW_P

        def load_idx(ci, sbuf, dbuf):
            pltpu.sync_copy(src_hbm.at[pl.ds(base_w + ci * CHP, CHP)], sbuf)
            pltpu.sync_copy(dst_hbm.at[pl.ds(base_w + ci * CHP, CHP)], dbuf)

        @pl.loop(0, NCHUNK_P, step=2)
        def _(i):
            load_idx(i, sa, da)
            pltpu.async_copy(y_hbm.at[sa], ra, gsa).wait()
            pltpu.sync_copy(ra, acc.at[da], add=True)
            load_idx(i + 1, sb, db)
            pltpu.async_copy(y_hbm.at[sb], rb, gsb).wait()
            pltpu.sync_copy(rb, acc.at[db], add=True)

        plsc.subcore_barrier()
        pltpu.sync_copy(acc.at[pl.ds(s * RPS_P, RPS_P)], out_hbm.at[c, s])

    return k(src_pad, dst_pad, y, zero_rows).reshape(NC, ACC_N, d)


_RB = 2000


def _tc_matmul(x, w):
    m, k = x.shape
    n = w.shape[1]

    def body(x_ref, w_ref, o_ref):
        o_ref[...] = jnp.dot(x_ref[...], w_ref[...],
                             preferred_element_type=jnp.float32)

    return pl.pallas_call(
        body,
        grid=(m // _RB,),
        in_specs=[pl.BlockSpec((_RB, k), lambda i: (i, 0)),
                  pl.BlockSpec((k, n), lambda i: (0, 0))],
        out_specs=pl.BlockSpec((_RB, n), lambda i: (i, 0)),
        out_shape=jax.ShapeDtypeStruct((m, n), jnp.float32),
    )(x, w)


def _tc_scale(xw, degp):
    def body(xw_ref, deg_ref, y_ref, dinv_ref):
        cnt = deg_ref[0, :, 0:1] + deg_ref[1, :, 0:1]
        dinv = lax.rsqrt(cnt + 1.0)
        dinv_ref[...] = dinv
        y_ref[...] = dinv * xw_ref[...]

    return pl.pallas_call(
        body,
        grid=(N // _RB,),
        in_specs=[pl.BlockSpec((_RB, D_H), lambda i: (i, 0)),
                  pl.BlockSpec((NC, _RB, DEGW), lambda i: (0, i, 0))],
        out_specs=[pl.BlockSpec((_RB, D_H), lambda i: (i, 0)),
                   pl.BlockSpec((_RB, 1), lambda i: (i, 0))],
        out_shape=[jax.ShapeDtypeStruct((N, D_H), jnp.float32),
                   jax.ShapeDtypeStruct((N, 1), jnp.float32)],
    )(xw, degp)


def _tc_hidden(p, y1, dinv, b1):
    def body(p_ref, y1_ref, dinv_ref, b1_ref, g_ref):
        agg = p_ref[0] + p_ref[1] + y1_ref[...]
        h = jnp.maximum(dinv_ref[...] * agg + b1_ref[...], 0.0)
        g_ref[...] = dinv_ref[...] * h

    return pl.pallas_call(
        body,
        grid=(N // _RB,),
        in_specs=[pl.BlockSpec((NC, _RB, D_H), lambda i: (0, i, 0)),
                  pl.BlockSpec((_RB, D_H), lambda i: (i, 0)),
                  pl.BlockSpec((_RB, 1), lambda i: (i, 0)),
                  pl.BlockSpec((1, D_H), lambda i: (0, 0))],
        out_specs=pl.BlockSpec((_RB, D_H), lambda i: (i, 0)),
        out_shape=jax.ShapeDtypeStruct((N, D_H), jnp.float32),
    )(p, y1, dinv, b1)


def _tc_final(q, g, dinv, w2, b2):
    def body(q_ref, g_ref, dinv_ref, w2_ref, b2_ref, o_ref):
        agg = dinv_ref[...] * (q_ref[0] + q_ref[1] + g_ref[...])
        o = jnp.dot(agg, w2_ref[...],
                    preferred_element_type=jnp.float32) + b2_ref[...]
        m = jnp.max(o, axis=1, keepdims=True)
        lse = jnp.log(jnp.sum(jnp.exp(o - m), axis=1, keepdims=True)) + m
        o_ref[...] = o - lse

    return pl.pallas_call(
        body,
        grid=(N // _RB,),
        in_specs=[pl.BlockSpec((NC, _RB, D_H), lambda i: (0, i, 0)),
                  pl.BlockSpec((_RB, D_H), lambda i: (i, 0)),
                  pl.BlockSpec((_RB, 1), lambda i: (i, 0)),
                  pl.BlockSpec((D_H, D_OUT), lambda i: (0, 0)),
                  pl.BlockSpec((1, D_OUT), lambda i: (0, 0))],
        out_specs=pl.BlockSpec((_RB, D_OUT), lambda i: (i, 0)),
        out_shape=jax.ShapeDtypeStruct((N, D_OUT), jnp.float32),
    )(q, g, dinv, w2, b2)


def kernel(x, edge_index, W1, b1, W2, b2):
    src = edge_index[0]
    dst = edge_index[1]
    pad = E_P - E
    src_pad = jnp.concatenate([src, jnp.zeros((pad,), jnp.int32)])
    dst_pad = jnp.concatenate([dst, jnp.full((pad,), N, jnp.int32)])
    b1r = b1.reshape(1, D_H)
    b2r = b2.reshape(1, D_OUT)

    degp = _sc_degree(dst)
    xw1 = _tc_matmul(x, W1)
    y1, dinv = _tc_scale(xw1, degp)
    p = _sc_aggregate(src_pad, dst_pad, y1, D_H)
    g = _tc_hidden(p, y1, dinv, b1r)
    q = _sc_aggregate(src_pad, dst_pad, g, D_H)
    return _tc_final(q, g, dinv, W2, b2r)


# speedup vs baseline: 1.7115x; 1.7115x over previous
import functools

import jax
import jax.numpy as jnp
from jax import lax
from jax.experimental import pallas as pl
from jax.experimental.pallas import tpu as pltpu
from jax.experimental.pallas import tpu_sc as plsc

N = 10000
E = 320000
D_IN = 128
D_H = 128
D_OUT = 64

NC = 2
NS = 16
NW = NC * NS
EPW = E // NW
CH = 80
NCHUNK = EPW // CH
RPS = N // NS
DEGW = 128

@functools.cache
def _mesh():
    return plsc.VectorSubcoreMesh(
        core_axis_name="c", subcore_axis_name="s",
        num_cores=NC, num_subcores=NS)


def _sc_degree(dst):
    ones_rows = jnp.ones((CH, DEGW), jnp.float32)
    zero_rows = jnp.zeros((RPS, DEGW), jnp.float32)

    @functools.partial(
        pl.kernel,
        out_type=jax.ShapeDtypeStruct((NC, NS, RPS, DEGW), jnp.float32),
        mesh=_mesh(),
        scratch_types=[
            pltpu.VMEM((CH,), jnp.int32),
            pltpu.VMEM((CH, DEGW), jnp.float32),
            pltpu.VMEM_SHARED((N, DEGW), jnp.float32),
        ],
    )
    def k(dst_hbm, ones_hbm, zero_hbm, out_hbm, dstv, onesv, acc):
        c = lax.axis_index("c")
        s = lax.axis_index("s")
        pltpu.sync_copy(zero_hbm, acc.at[pl.ds(s * RPS, RPS)])
        pltpu.sync_copy(ones_hbm, onesv)
        plsc.subcore_barrier()
        base_w = (c * NS + s) * EPW

        @pl.loop(0, NCHUNK)
        def _(ci):
            pltpu.sync_copy(dst_hbm.at[pl.ds(base_w + ci * CH, CH)], dstv)
            pltpu.sync_copy(onesv, acc.at[dstv], add=True)

        plsc.subcore_barrier()
        pltpu.sync_copy(acc.at[pl.ds(s * RPS, RPS)], out_hbm.at[c, s])

    return k(dst, ones_rows, zero_rows).reshape(NC, N, DEGW)


CHP = 80
NCHUNK_P = 126
EPW_P = CHP * NCHUNK_P
E_P = NW * EPW_P
ACC_N = 10080
RPS_P = ACC_N // NS


def _sc_aggregate(src_pad, dst_pad, y, d):
    zero_rows = jnp.zeros((RPS_P, d), jnp.float32)

    @functools.partial(
        pl.kernel,
        out_type=jax.ShapeDtypeStruct((NC, NS, RPS_P, d), jnp.float32),
        mesh=_mesh(),
        scratch_types=[
            pltpu.VMEM((CHP,), jnp.int32),
            pltpu.VMEM((CHP,), jnp.int32),
            pltpu.VMEM((CHP, d), jnp.float32),
            pltpu.VMEM((CHP,), jnp.int32),
            pltpu.VMEM((CHP,), jnp.int32),
            pltpu.VMEM((CHP, d), jnp.float32),
            pltpu.VMEM_SHARED((ACC_N, d), jnp.float32),
            pltpu.SemaphoreType.DMA,
            pltpu.SemaphoreType.DMA,
        ],
    )
    def k(src_hbm, dst_hbm, y_hbm, zero_hbm, out_hbm,
          sa, da, ra, sb, db, rb, acc, gsa, gsb):
        c = lax.axis_index("c")
        s = lax.axis_index("s")
        pltpu.sync_copy(zero_hbm, acc.at[pl.ds(s * RPS_P, RPS_P)])
        plsc.subcore_barrier()
        base_w = (c * NS + s) * EPW_P

        def load_idx(ci, sbuf, dbuf):
            pltpu.sync_copy(src_hbm.at[pl.ds(base_w + ci * CHP, CHP)], sbuf)
            pltpu.sync_copy(dst_hbm.at[pl.ds(base_w + ci * CHP, CHP)], dbuf)

        load_idx(0, sa, da)
        pltpu.async_copy(y_hbm.at[sa], ra, gsa)
        load_idx(1, sb, db)
        pltpu.async_copy(y_hbm.at[sb], rb, gsb)

        @pl.loop(0, NCHUNK_P, step=2)
        def _(i):
            pltpu.make_async_copy(y_hbm.at[sa], ra, gsa).wait()
            pltpu.sync_copy(ra, acc.at[da], add=True)

            @pl.when(i + 2 < NCHUNK_P)
            def _():
                load_idx(i + 2, sa, da)
                pltpu.async_copy(y_hbm.at[sa], ra, gsa)

            pltpu.make_async_copy(y_hbm.at[sb], rb, gsb).wait()
            pltpu.sync_copy(rb, acc.at[db], add=True)

            @pl.when(i + 3 < NCHUNK_P)
            def _():
                load_idx(i + 3, sb, db)
                pltpu.async_copy(y_hbm.at[sb], rb, gsb)

        plsc.subcore_barrier()
        pltpu.sync_copy(acc.at[pl.ds(s * RPS_P, RPS_P)], out_hbm.at[c, s])

    return k(src_pad, dst_pad, y, zero_rows).reshape(NC, ACC_N, d)


_RB = 2000


def _tc_matmul(x, w):
    m, k = x.shape
    n = w.shape[1]

    def body(x_ref, w_ref, o_ref):
        o_ref[...] = jnp.dot(x_ref[...], w_ref[...],
                             preferred_element_type=jnp.float32)

    return pl.pallas_call(
        body,
        grid=(m // _RB,),
        in_specs=[pl.BlockSpec((_RB, k), lambda i: (i, 0)),
                  pl.BlockSpec((k, n), lambda i: (0, 0))],
        out_specs=pl.BlockSpec((_RB, n), lambda i: (i, 0)),
        out_shape=jax.ShapeDtypeStruct((m, n), jnp.float32),
    )(x, w)


def _tc_scale(xw, degp):
    def body(xw_ref, deg_ref, y_ref, dinv_ref):
        cnt = deg_ref[0, :, 0:1] + deg_ref[1, :, 0:1]
        dinv = lax.rsqrt(cnt + 1.0)
        dinv_ref[...] = dinv
        y_ref[...] = dinv * xw_ref[...]

    return pl.pallas_call(
        body,
        grid=(N // _RB,),
        in_specs=[pl.BlockSpec((_RB, D_H), lambda i: (i, 0)),
                  pl.BlockSpec((NC, _RB, DEGW), lambda i: (0, i, 0))],
        out_specs=[pl.BlockSpec((_RB, D_H), lambda i: (i, 0)),
                   pl.BlockSpec((_RB, 1), lambda i: (i, 0))],
        out_shape=[jax.ShapeDtypeStruct((N, D_H), jnp.float32),
                   jax.ShapeDtypeStruct((N, 1), jnp.float32)],
    )(xw, degp)


def _tc_hidden(p, y1, dinv, b1):
    def body(p_ref, y1_ref, dinv_ref, b1_ref, g_ref):
        agg = p_ref[0] + p_ref[1] + y1_ref[...]
        h = jnp.maximum(dinv_ref[...] * agg + b1_ref[...], 0.0)
        g_ref[...] = dinv_ref[...] * h

    return pl.pallas_call(
        body,
        grid=(N // _RB,),
        in_specs=[pl.BlockSpec((NC, _RB, D_H), lambda i: (0, i, 0)),
                  pl.BlockSpec((_RB, D_H), lambda i: (i, 0)),
                  pl.BlockSpec((_RB, 1), lambda i: (i, 0)),
                  pl.BlockSpec((1, D_H), lambda i: (0, 0))],
        out_specs=pl.BlockSpec((_RB, D_H), lambda i: (i, 0)),
        out_shape=jax.ShapeDtypeStruct((N, D_H), jnp.float32),
    )(p, y1, dinv, b1)


def _tc_final(q, g, dinv, w2, b2):
    def body(q_ref, g_ref, dinv_ref, w2_ref, b2_ref, o_ref):
        agg = dinv_ref[...] * (q_ref[0] + q_ref[1] + g_ref[...])
        o = jnp.dot(agg, w2_ref[...],
                    preferred_element_type=jnp.float32) + b2_ref[...]
        m = jnp.max(o, axis=1, keepdims=True)
        lse = jnp.log(jnp.sum(jnp.exp(o - m), axis=1, keepdims=True)) + m
        o_ref[...] = o - lse

    return pl.pallas_call(
        body,
        grid=(N // _RB,),
        in_specs=[pl.BlockSpec((NC, _RB, D_H), lambda i: (0, i, 0)),
                  pl.BlockSpec((_RB, D_H), lambda i: (i, 0)),
                  pl.BlockSpec((_RB, 1), lambda i: (i, 0)),
                  pl.BlockSpec((D_H, D_OUT), lambda i: (0, 0)),
                  pl.BlockSpec((1, D_OUT), lambda i: (0, 0))],
        out_specs=pl.BlockSpec((_RB, D_OUT), lambda i: (i, 0)),
        out_shape=jax.ShapeDtypeStruct((N, D_OUT), jnp.float32),
    )(q, g, dinv, w2, b2)


def kernel(x, edge_index, W1, b1, W2, b2):
    src = edge_index[0]
    dst = edge_index[1]
    pad = E_P - E
    src_pad = jnp.concatenate([src, jnp.zeros((pad,), jnp.int32)])
    dst_pad = jnp.concatenate([dst, jnp.full((pad,), N, jnp.int32)])
    b1r = b1.reshape(1, D_H)
    b2r = b2.reshape(1, D_OUT)

    degp = _sc_degree(dst)
    xw1 = _tc_matmul(x, W1)
    y1, dinv = _tc_scale(xw1, degp)
    p = _sc_aggregate(src_pad, dst_pad, y1, D_H)
    g = _tc_hidden(p, y1, dinv, b1r)
    q = _sc_aggregate(src_pad, dst_pad, g, D_H)
    return _tc_final(q, g, dinv, W2, b2r)


# speedup vs baseline: 1.7937x; 1.0481x over previous
import functools

import jax
import jax.numpy as jnp
from jax import lax
from jax.experimental import pallas as pl
from jax.experimental.pallas import tpu as pltpu
from jax.experimental.pallas import tpu_sc as plsc

N = 10000
E = 320000
D_IN = 128
D_H = 128
D_OUT = 64

NC = 2
NS = 16
NW = NC * NS
EPW = E // NW
CH = 80
NCHUNK = EPW // CH
RPS = N // NS
DEGW = 128

@functools.cache
def _mesh():
    return plsc.VectorSubcoreMesh(
        core_axis_name="c", subcore_axis_name="s",
        num_cores=NC, num_subcores=NS)


def _sc_degree(dst):
    ones_rows = jnp.ones((CH, DEGW), jnp.float32)
    zero_rows = jnp.zeros((RPS, DEGW), jnp.float32)

    @functools.partial(
        pl.kernel,
        out_type=jax.ShapeDtypeStruct((NC, NS, RPS, DEGW), jnp.float32),
        mesh=_mesh(),
        scratch_types=[
            pltpu.VMEM((CH,), jnp.int32),
            pltpu.VMEM((CH, DEGW), jnp.float32),
            pltpu.VMEM_SHARED((N, DEGW), jnp.float32),
        ],
    )
    def k(dst_hbm, ones_hbm, zero_hbm, out_hbm, dstv, onesv, acc):
        c = lax.axis_index("c")
        s = lax.axis_index("s")
        pltpu.sync_copy(zero_hbm, acc.at[pl.ds(s * RPS, RPS)])
        pltpu.sync_copy(ones_hbm, onesv)
        plsc.subcore_barrier()
        base_w = (c * NS + s) * EPW

        @pl.loop(0, NCHUNK)
        def _(ci):
            pltpu.sync_copy(dst_hbm.at[pl.ds(base_w + ci * CH, CH)], dstv)
            pltpu.sync_copy(onesv, acc.at[dstv], add=True)

        plsc.subcore_barrier()
        pltpu.sync_copy(acc.at[pl.ds(s * RPS, RPS)], out_hbm.at[c, s])

    return k(dst, ones_rows, zero_rows).reshape(NC, N, DEGW)


CHP = 112
NCHUNK_P = 90
EPW_P = CHP * NCHUNK_P
E_P = NW * EPW_P
ACC_N = 10080
RPS_P = ACC_N // NS


def _sc_aggregate(src_pad, dst_pad, y, d):
    zero_rows = jnp.zeros((RPS_P, d), jnp.float32)

    @functools.partial(
        pl.kernel,
        out_type=jax.ShapeDtypeStruct((NC, NS, RPS_P, d), jnp.float32),
        mesh=_mesh(),
        scratch_types=[
            pltpu.VMEM((CHP,), jnp.int32),
            pltpu.VMEM((CHP,), jnp.int32),
            pltpu.VMEM((CHP, d), jnp.float32),
            pltpu.VMEM((CHP,), jnp.int32),
            pltpu.VMEM((CHP,), jnp.int32),
            pltpu.VMEM((CHP, d), jnp.float32),
            pltpu.VMEM_SHARED((ACC_N, d), jnp.float32),
            pltpu.SemaphoreType.DMA,
            pltpu.SemaphoreType.DMA,
        ],
    )
    def k(src_hbm, dst_hbm, y_hbm, zero_hbm, out_hbm,
          sa, da, ra, sb, db, rb, acc, gsa, gsb):
        c = lax.axis_index("c")
        s = lax.axis_index("s")
        pltpu.sync_copy(zero_hbm, acc.at[pl.ds(s * RPS_P, RPS_P)])
        plsc.subcore_barrier()
        base_w = (c * NS + s) * EPW_P

        def load_idx(ci, sbuf, dbuf):
            pltpu.sync_copy(src_hbm.at[pl.ds(base_w + ci * CHP, CHP)], sbuf)
            pltpu.sync_copy(dst_hbm.at[pl.ds(base_w + ci * CHP, CHP)], dbuf)

        load_idx(0, sa, da)
        pltpu.async_copy(y_hbm.at[sa], ra, gsa)
        load_idx(1, sb, db)
        pltpu.async_copy(y_hbm.at[sb], rb, gsb)

        @pl.loop(0, NCHUNK_P, step=2)
        def _(i):
            pltpu.make_async_copy(y_hbm.at[sa], ra, gsa).wait()
            pltpu.sync_copy(ra, acc.at[da], add=True)

            @pl.when(i + 2 < NCHUNK_P)
            def _():
                load_idx(i + 2, sa, da)
                pltpu.async_copy(y_hbm.at[sa], ra, gsa)

            pltpu.make_async_copy(y_hbm.at[sb], rb, gsb).wait()
            pltpu.sync_copy(rb, acc.at[db], add=True)

            @pl.when(i + 3 < NCHUNK_P)
            def _():
                load_idx(i + 3, sb, db)
                pltpu.async_copy(y_hbm.at[sb], rb, gsb)

        plsc.subcore_barrier()
        pltpu.sync_copy(acc.at[pl.ds(s * RPS_P, RPS_P)], out_hbm.at[c, s])

    return k(src_pad, dst_pad, y, zero_rows).reshape(NC, ACC_N, d)


_RB = 2000


def _tc_matmul(x, w):
    m, k = x.shape
    n = w.shape[1]

    def body(x_ref, w_ref, o_ref):
        o_ref[...] = jnp.dot(x_ref[...], w_ref[...],
                             preferred_element_type=jnp.float32)

    return pl.pallas_call(
        body,
        grid=(m // _RB,),
        in_specs=[pl.BlockSpec((_RB, k), lambda i: (i, 0)),
                  pl.BlockSpec((k, n), lambda i: (0, 0))],
        out_specs=pl.BlockSpec((_RB, n), lambda i: (i, 0)),
        out_shape=jax.ShapeDtypeStruct((m, n), jnp.float32),
    )(x, w)


def _tc_scale(xw, degp):
    def body(xw_ref, deg_ref, y_ref, dinv_ref):
        cnt = deg_ref[0, :, 0:1] + deg_ref[1, :, 0:1]
        dinv = lax.rsqrt(cnt + 1.0)
        dinv_ref[...] = dinv
        y_ref[...] = dinv * xw_ref[...]

    return pl.pallas_call(
        body,
        grid=(N // _RB,),
        in_specs=[pl.BlockSpec((_RB, D_H), lambda i: (i, 0)),
                  pl.BlockSpec((NC, _RB, DEGW), lambda i: (0, i, 0))],
        out_specs=[pl.BlockSpec((_RB, D_H), lambda i: (i, 0)),
                   pl.BlockSpec((_RB, 1), lambda i: (i, 0))],
        out_shape=[jax.ShapeDtypeStruct((N, D_H), jnp.float32),
                   jax.ShapeDtypeStruct((N, 1), jnp.float32)],
    )(xw, degp)


def _tc_hidden(p, y1, dinv, b1):
    def body(p_ref, y1_ref, dinv_ref, b1_ref, g_ref):
        agg = p_ref[0] + p_ref[1] + y1_ref[...]
        h = jnp.maximum(dinv_ref[...] * agg + b1_ref[...], 0.0)
        g_ref[...] = dinv_ref[...] * h

    return pl.pallas_call(
        body,
        grid=(N // _RB,),
        in_specs=[pl.BlockSpec((NC, _RB, D_H), lambda i: (0, i, 0)),
                  pl.BlockSpec((_RB, D_H), lambda i: (i, 0)),
                  pl.BlockSpec((_RB, 1), lambda i: (i, 0)),
                  pl.BlockSpec((1, D_H), lambda i: (0, 0))],
        out_specs=pl.BlockSpec((_RB, D_H), lambda i: (i, 0)),
        out_shape=jax.ShapeDtypeStruct((N, D_H), jnp.float32),
    )(p, y1, dinv, b1)


def _tc_final(q, g, dinv, w2, b2):
    def body(q_ref, g_ref, dinv_ref, w2_ref, b2_ref, o_ref):
        agg = dinv_ref[...] * (q_ref[0] + q_ref[1] + g_ref[...])
        o = jnp.dot(agg, w2_ref[...],
                    preferred_element_type=jnp.float32) + b2_ref[...]
        m = jnp.max(o, axis=1, keepdims=True)
        lse = jnp.log(jnp.sum(jnp.exp(o - m), axis=1, keepdims=True)) + m
        o_ref[...] = o - lse

    return pl.pallas_call(
        body,
        grid=(N // _RB,),
        in_specs=[pl.BlockSpec((NC, _RB, D_H), lambda i: (0, i, 0)),
                  pl.BlockSpec((_RB, D_H), lambda i: (i, 0)),
                  pl.BlockSpec((_RB, 1), lambda i: (i, 0)),
                  pl.BlockSpec((D_H, D_OUT), lambda i: (0, 0)),
                  pl.BlockSpec((1, D_OUT), lambda i: (0, 0))],
        out_specs=pl.BlockSpec((_RB, D_OUT), lambda i: (i, 0)),
        out_shape=jax.ShapeDtypeStruct((N, D_OUT), jnp.float32),
    )(q, g, dinv, w2, b2)


def kernel(x, edge_index, W1, b1, W2, b2):
    src = edge_index[0]
    dst = edge_index[1]
    pad = E_P - E
    src_pad = jnp.concatenate([src, jnp.zeros((pad,), jnp.int32)])
    dst_pad = jnp.concatenate([dst, jnp.full((pad,), N, jnp.int32)])
    b1r = b1.reshape(1, D_H)
    b2r = b2.reshape(1, D_OUT)

    degp = _sc_degree(dst)
    xw1 = _tc_matmul(x, W1)
    y1, dinv = _tc_scale(xw1, degp)
    p = _sc_aggregate(src_pad, dst_pad, y1, D_H)
    g = _tc_hidden(p, y1, dinv, b1r)
    q = _sc_aggregate(src_pad, dst_pad, g, D_H)
    return _tc_final(q, g, dinv, W2, b2r)


# speedup vs baseline: 1.8088x; 1.0084x over previous
import functools

import jax
import jax.numpy as jnp
from jax import lax
from jax.experimental import pallas as pl
from jax.experimental.pallas import tpu as pltpu
from jax.experimental.pallas import tpu_sc as plsc

N = 10000
E = 320000
D_IN = 128
D_H = 128
D_OUT = 64

NC = 2
NS = 16
NW = NC * NS
EPW = E // NW
CH = 80
NCHUNK = EPW // CH
RPS = N // NS
DEGW = 128

@functools.cache
def _mesh():
    return plsc.VectorSubcoreMesh(
        core_axis_name="c", subcore_axis_name="s",
        num_cores=NC, num_subcores=NS)


def _sc_degree(dst):
    ones_rows = jnp.ones((CH, DEGW), jnp.float32)
    zero_rows = jnp.zeros((RPS, DEGW), jnp.float32)

    @functools.partial(
        pl.kernel,
        out_type=jax.ShapeDtypeStruct((NC, NS, RPS, DEGW), jnp.float32),
        mesh=_mesh(),
        scratch_types=[
            pltpu.VMEM((CH,), jnp.int32),
            pltpu.VMEM((CH, DEGW), jnp.float32),
            pltpu.VMEM_SHARED((N, DEGW), jnp.float32),
        ],
    )
    def k(dst_hbm, ones_hbm, zero_hbm, out_hbm, dstv, onesv, acc):
        c = lax.axis_index("c")
        s = lax.axis_index("s")
        pltpu.sync_copy(zero_hbm, acc.at[pl.ds(s * RPS, RPS)])
        pltpu.sync_copy(ones_hbm, onesv)
        plsc.subcore_barrier()
        base_w = (c * NS + s) * EPW

        @pl.loop(0, NCHUNK)
        def _(ci):
            pltpu.sync_copy(dst_hbm.at[pl.ds(base_w + ci * CH, CH)], dstv)
            pltpu.sync_copy(onesv, acc.at[dstv], add=True)

        plsc.subcore_barrier()
        pltpu.sync_copy(acc.at[pl.ds(s * RPS, RPS)], out_hbm.at[c, s])

    return k(dst, ones_rows, zero_rows).reshape(NC, N, DEGW)


CHP = 120
NCHUNK_P = 84
EPW_P = CHP * NCHUNK_P
E_P = NW * EPW_P
ACC_N = 10080
RPS_P = ACC_N // NS


def _sc_aggregate(src_pad, dst_pad, y, d):
    zero_rows = jnp.zeros((RPS_P, d), jnp.float32)

    @functools.partial(
        pl.kernel,
        out_type=jax.ShapeDtypeStruct((NC, NS, RPS_P, d), jnp.float32),
        mesh=_mesh(),
        scratch_types=[
            pltpu.VMEM((CHP,), jnp.int32),
            pltpu.VMEM((CHP,), jnp.int32),
            pltpu.VMEM((CHP, d), jnp.float32),
            pltpu.VMEM((CHP,), jnp.int32),
            pltpu.VMEM((CHP,), jnp.int32),
            pltpu.VMEM((CHP, d), jnp.float32),
            pltpu.VMEM_SHARED((ACC_N, d), jnp.float32),
            pltpu.SemaphoreType.DMA,
            pltpu.SemaphoreType.DMA,
        ],
    )
    def k(src_hbm, dst_hbm, y_hbm, zero_hbm, out_hbm,
          sa, da, ra, sb, db, rb, acc, gsa, gsb):
        c = lax.axis_index("c")
        s = lax.axis_index("s")
        pltpu.sync_copy(zero_hbm, acc.at[pl.ds(s * RPS_P, RPS_P)])
        plsc.subcore_barrier()
        base_w = (c * NS + s) * EPW_P

        def load_idx(ci, sbuf, dbuf):
            pltpu.sync_copy(src_hbm.at[pl.ds(base_w + ci * CHP, CHP)], sbuf)
            pltpu.sync_copy(dst_hbm.at[pl.ds(base_w + ci * CHP, CHP)], dbuf)

        load_idx(0, sa, da)
        pltpu.async_copy(y_hbm.at[sa], ra, gsa)
        load_idx(1, sb, db)
        pltpu.async_copy(y_hbm.at[sb], rb, gsb)

        @pl.loop(0, NCHUNK_P, step=2)
        def _(i):
            pltpu.make_async_copy(y_hbm.at[sa], ra, gsa).wait()
            pltpu.sync_copy(ra, acc.at[da], add=True)

            @pl.when(i + 2 < NCHUNK_P)
            def _():
                load_idx(i + 2, sa, da)
                pltpu.async_copy(y_hbm.at[sa], ra, gsa)

            pltpu.make_async_copy(y_hbm.at[sb], rb, gsb).wait()
            pltpu.sync_copy(rb, acc.at[db], add=True)

            @pl.when(i + 3 < NCHUNK_P)
            def _():
                load_idx(i + 3, sb, db)
                pltpu.async_copy(y_hbm.at[sb], rb, gsb)

        plsc.subcore_barrier()
        pltpu.sync_copy(acc.at[pl.ds(s * RPS_P, RPS_P)], out_hbm.at[c, s])

    return k(src_pad, dst_pad, y, zero_rows).reshape(NC, ACC_N, d)


_RB = 2000


def _tc_matmul(x, w):
    m, k = x.shape
    n = w.shape[1]

    def body(x_ref, w_ref, o_ref):
        o_ref[...] = jnp.dot(x_ref[...], w_ref[...],
                             preferred_element_type=jnp.float32)

    return pl.pallas_call(
        body,
        grid=(m // _RB,),
        in_specs=[pl.BlockSpec((_RB, k), lambda i: (i, 0)),
                  pl.BlockSpec((k, n), lambda i: (0, 0))],
        out_specs=pl.BlockSpec((_RB, n), lambda i: (i, 0)),
        out_shape=jax.ShapeDtypeStruct((m, n), jnp.float32),
    )(x, w)


def _tc_scale(xw, degp):
    def body(xw_ref, deg_ref, y_ref, dinv_ref):
        cnt = deg_ref[0, :, 0:1] + deg_ref[1, :, 0:1]
        dinv = lax.rsqrt(cnt + 1.0)
        dinv_ref[...] = dinv
        y_ref[...] = dinv * xw_ref[...]

    return pl.pallas_call(
        body,
        grid=(N // _RB,),
        in_specs=[pl.BlockSpec((_RB, D_H), lambda i: (i, 0)),
                  pl.BlockSpec((NC, _RB, DEGW), lambda i: (0, i, 0))],
        out_specs=[pl.BlockSpec((_RB, D_H), lambda i: (i, 0)),
                   pl.BlockSpec((_RB, 1), lambda i: (i, 0))],
        out_shape=[jax.ShapeDtypeStruct((N, D_H), jnp.float32),
                   jax.ShapeDtypeStruct((N, 1), jnp.float32)],
    )(xw, degp)


def _tc_hidden(p, y1, dinv, b1):
    def body(p_ref, y1_ref, dinv_ref, b1_ref, g_ref):
        agg = p_ref[0] + p_ref[1] + y1_ref[...]
        h = jnp.maximum(dinv_ref[...] * agg + b1_ref[...], 0.0)
        g_ref[...] = dinv_ref[...] * h

    return pl.pallas_call(
        body,
        grid=(N // _RB,),
        in_specs=[pl.BlockSpec((NC, _RB, D_H), lambda i: (0, i, 0)),
                  pl.BlockSpec((_RB, D_H), lambda i: (i, 0)),
                  pl.BlockSpec((_RB, 1), lambda i: (i, 0)),
                  pl.BlockSpec((1, D_H), lambda i: (0, 0))],
        out_specs=pl.BlockSpec((_RB, D_H), lambda i: (i, 0)),
        out_shape=jax.ShapeDtypeStruct((N, D_H), jnp.float32),
    )(p, y1, dinv, b1)


def _tc_final(q, g, dinv, w2, b2):
    def body(q_ref, g_ref, dinv_ref, w2_ref, b2_ref, o_ref):
        agg = dinv_ref[...] * (q_ref[0] + q_ref[1] + g_ref[...])
        o = jnp.dot(agg, w2_ref[...],
                    preferred_element_type=jnp.float32) + b2_ref[...]
        m = jnp.max(o, axis=1, keepdims=True)
        lse = jnp.log(jnp.sum(jnp.exp(o - m), axis=1, keepdims=True)) + m
        o_ref[...] = o - lse

    return pl.pallas_call(
        body,
        grid=(N // _RB,),
        in_specs=[pl.BlockSpec((NC, _RB, D_H), lambda i: (0, i, 0)),
                  pl.BlockSpec((_RB, D_H), lambda i: (i, 0)),
                  pl.BlockSpec((_RB, 1), lambda i: (i, 0)),
                  pl.BlockSpec((D_H, D_OUT), lambda i: (0, 0)),
                  pl.BlockSpec((1, D_OUT), lambda i: (0, 0))],
        out_specs=pl.BlockSpec((_RB, D_OUT), lambda i: (i, 0)),
        out_shape=jax.ShapeDtypeStruct((N, D_OUT), jnp.float32),
    )(q, g, dinv, w2, b2)


def kernel(x, edge_index, W1, b1, W2, b2):
    src = edge_index[0]
    dst = edge_index[1]
    pad = E_P - E
    src_pad = jnp.concatenate([src, jnp.zeros((pad,), jnp.int32)])
    dst_pad = jnp.concatenate([dst, jnp.full((pad,), N, jnp.int32)])
    b1r = b1.reshape(1, D_H)
    b2r = b2.reshape(1, D_OUT)

    degp = _sc_degree(dst)
    xw1 = _tc_matmul(x, W1)
    y1, dinv = _tc_scale(xw1, degp)
    p = _sc_aggregate(src_pad, dst_pad, y1, D_H)
    g = _tc_hidden(p, y1, dinv, b1r)
    q = _sc_aggregate(src_pad, dst_pad, g, D_H)
    return _tc_final(q, g, dinv, W2, b2r)


# speedup vs baseline: 1.8460x; 1.0206x over previous
import functools

import jax
import jax.numpy as jnp
from jax import lax
from jax.experimental import pallas as pl
from jax.experimental.pallas import tpu as pltpu
from jax.experimental.pallas import tpu_sc as plsc

N = 10000
E = 320000
D_IN = 128
D_H = 128
D_OUT = 64

NC = 2
NS = 16
NW = NC * NS
EPW = E // NW
CH = 80
NCHUNK = EPW // CH
RPS = N // NS
DEGW = 128

@functools.cache
def _mesh():
    return plsc.VectorSubcoreMesh(
        core_axis_name="c", subcore_axis_name="s",
        num_cores=NC, num_subcores=NS)


def _sc_degree(dst):
    ones_rows = jnp.ones((CH, DEGW), jnp.float32)
    zero_rows = jnp.zeros((RPS, DEGW), jnp.float32)

    @functools.partial(
        pl.kernel,
        out_type=jax.ShapeDtypeStruct((NC, NS, RPS, DEGW), jnp.float32),
        mesh=_mesh(),
        scratch_types=[
            pltpu.VMEM((CH,), jnp.int32),
            pltpu.VMEM((CH, DEGW), jnp.float32),
            pltpu.VMEM_SHARED((N, DEGW), jnp.float32),
        ],
    )
    def k(dst_hbm, ones_hbm, zero_hbm, out_hbm, dstv, onesv, acc):
        c = lax.axis_index("c")
        s = lax.axis_index("s")
        pltpu.sync_copy(zero_hbm, acc.at[pl.ds(s * RPS, RPS)])
        pltpu.sync_copy(ones_hbm, onesv)
        plsc.subcore_barrier()
        base_w = (c * NS + s) * EPW

        @pl.loop(0, NCHUNK)
        def _(ci):
            pltpu.sync_copy(dst_hbm.at[pl.ds(base_w + ci * CH, CH)], dstv)
            pltpu.sync_copy(onesv, acc.at[dstv], add=True)

        plsc.subcore_barrier()
        pltpu.sync_copy(acc.at[pl.ds(s * RPS, RPS)], out_hbm.at[c, s])

    return k(dst, ones_rows, zero_rows).reshape(NC, N, DEGW)


CHP = 120
NCHUNK_P = 84
EPW_P = CHP * NCHUNK_P
E_P = NW * EPW_P
ACC_N = 10080
RPS_P = ACC_N // NS


def _sc_aggregate(src_pad, dst_pad, y, d):
    zero_rows = jnp.zeros((RPS_P, d), jnp.float32)

    @functools.partial(
        pl.kernel,
        out_type=jax.ShapeDtypeStruct((NC, NS, RPS_P, d), jnp.float32),
        mesh=_mesh(),
        scratch_types=[
            pltpu.VMEM((CHP,), jnp.int32),
            pltpu.VMEM((CHP,), jnp.int32),
            pltpu.VMEM((CHP, d), jnp.float32),
            pltpu.VMEM((CHP,), jnp.int32),
            pltpu.VMEM((CHP,), jnp.int32),
            pltpu.VMEM((CHP, d), jnp.float32),
            pltpu.VMEM_SHARED((ACC_N, d), jnp.float32),
            pltpu.SemaphoreType.DMA,
            pltpu.SemaphoreType.DMA,
        ],
    )
    def k(src_hbm, dst_hbm, y_hbm, zero_hbm, out_hbm,
          sa, da, ra, sb, db, rb, acc, gsa, gsb):
        c = lax.axis_index("c")
        s = lax.axis_index("s")
        pltpu.sync_copy(zero_hbm, acc.at[pl.ds(s * RPS_P, RPS_P)])
        plsc.subcore_barrier()
        base_w = (c * NS + s) * EPW_P

        def load_idx(ci, sbuf, dbuf):
            pltpu.sync_copy(src_hbm.at[pl.ds(base_w + ci * CHP, CHP)], sbuf)
            pltpu.sync_copy(dst_hbm.at[pl.ds(base_w + ci * CHP, CHP)], dbuf)

        load_idx(0, sa, da)
        pltpu.async_copy(y_hbm.at[sa], ra, gsa)
        load_idx(1, sb, db)
        pltpu.async_copy(y_hbm.at[sb], rb, gsb)

        @pl.loop(0, NCHUNK_P, step=2)
        def _(i):
            pltpu.make_async_copy(y_hbm.at[sa], ra, gsa).wait()
            pltpu.sync_copy(ra, acc.at[da], add=True)

            @pl.when(i + 2 < NCHUNK_P)
            def _():
                load_idx(i + 2, sa, da)
                pltpu.async_copy(y_hbm.at[sa], ra, gsa)

            pltpu.make_async_copy(y_hbm.at[sb], rb, gsb).wait()
            pltpu.sync_copy(rb, acc.at[db], add=True)

            @pl.when(i + 3 < NCHUNK_P)
            def _():
                load_idx(i + 3, sb, db)
                pltpu.async_copy(y_hbm.at[sb], rb, gsb)

        plsc.subcore_barrier()
        pltpu.sync_copy(acc.at[pl.ds(s * RPS_P, RPS_P)], out_hbm.at[c, s])

    return k(src_pad, dst_pad, y, zero_rows).reshape(NC, ACC_N, d)


_RB = 2000


def _tc_matmul(x, w):
    m, k = x.shape
    n = w.shape[1]

    def body(x_ref, w_ref, o_ref):
        o_ref[...] = jnp.dot(x_ref[...], w_ref[...],
                             preferred_element_type=jnp.float32)

    return pl.pallas_call(
        body,
        grid=(m // _RB,),
        in_specs=[pl.BlockSpec((_RB, k), lambda i: (i, 0)),
                  pl.BlockSpec((k, n), lambda i: (0, 0))],
        out_specs=pl.BlockSpec((_RB, n), lambda i: (i, 0)),
        out_shape=jax.ShapeDtypeStruct((m, n), jnp.float32),
    )(x, w)


def _tc_scale(xw, degp):
    def body(xw_ref, deg_ref, y_ref, dinv_ref):
        cnt = deg_ref[0, :, 0:1] + deg_ref[1, :, 0:1]
        dinv = lax.rsqrt(cnt + 1.0)
        dinv_ref[...] = dinv
        y_ref[...] = dinv * xw_ref[...]

    return pl.pallas_call(
        body,
        grid=(N // _RB,),
        in_specs=[pl.BlockSpec((_RB, D_H), lambda i: (i, 0)),
                  pl.BlockSpec((NC, _RB, DEGW), lambda i: (0, i, 0))],
        out_specs=[pl.BlockSpec((_RB, D_H), lambda i: (i, 0)),
                   pl.BlockSpec((_RB, 1), lambda i: (i, 0))],
        out_shape=[jax.ShapeDtypeStruct((N, D_H), jnp.float32),
                   jax.ShapeDtypeStruct((N, 1), jnp.float32)],
    )(xw, degp)


def _tc_hidden(p, y1, dinv, b1):
    def body(p_ref, y1_ref, dinv_ref, b1_ref, g_ref):
        agg = p_ref[0] + p_ref[1] + y1_ref[...]
        h = jnp.maximum(dinv_ref[...] * agg + b1_ref[...], 0.0)
        g_ref[...] = dinv_ref[...] * h

    return pl.pallas_call(
        body,
        grid=(N // _RB,),
        in_specs=[pl.BlockSpec((NC, _RB, D_H), lambda i: (0, i, 0)),
                  pl.BlockSpec((_RB, D_H), lambda i: (i, 0)),
                  pl.BlockSpec((_RB, 1), lambda i: (i, 0)),
                  pl.BlockSpec((1, D_H), lambda i: (0, 0))],
        out_specs=pl.BlockSpec((_RB, D_H), lambda i: (i, 0)),
        out_shape=jax.ShapeDtypeStruct((N, D_H), jnp.float32),
    )(p, y1, dinv, b1)


def _tc_final(q, g, dinv, w2, b2):
    def body(q_ref, g_ref, dinv_ref, w2_ref, b2_ref, o_ref):
        agg = dinv_ref[...] * (q_ref[0] + q_ref[1] + g_ref[...])
        o = jnp.dot(agg, w2_ref[...],
                    preferred_element_type=jnp.float32) + b2_ref[...]
        m = jnp.max(o, axis=1, keepdims=True)
        lse = jnp.log(jnp.sum(jnp.exp(o - m), axis=1, keepdims=True)) + m
        o_ref[...] = o - lse

    return pl.pallas_call(
        body,
        grid=(N // _RB,),
        in_specs=[pl.BlockSpec((NC, _RB, D_H), lambda i: (0, i, 0)),
                  pl.BlockSpec((_RB, D_H), lambda i: (i, 0)),
                  pl.BlockSpec((_RB, 1), lambda i: (i, 0)),
                  pl.BlockSpec((D_H, D_OUT), lambda i: (0, 0)),
                  pl.BlockSpec((1, D_OUT), lambda i: (0, 0))],
        out_specs=pl.BlockSpec((_RB, D_OUT), lambda i: (i, 0)),
        out_shape=jax.ShapeDtypeStruct((N, D_OUT), jnp.float32),
    )(q, g, dinv, w2, b2)


def kernel(x, edge_index, W1, b1, W2, b2):
    src = edge_index[0]
    dst = edge_index[1]
    pad = E_P - E
    src_pad = jnp.concatenate([src, jnp.zeros((pad,), jnp.int32)])
    fake_dst = N + (jnp.arange(pad, dtype=jnp.int32) % (ACC_N - N))
    dst_pad = jnp.concatenate([dst, fake_dst])
    b1r = b1.reshape(1, D_H)
    b2r = b2.reshape(1, D_OUT)

    degp = _sc_degree(dst)
    xw1 = _tc_matmul(x, W1)
    y1, dinv = _tc_scale(xw1, degp)
    p = _sc_aggregate(src_pad, dst_pad, y1, D_H)
    g = _tc_hidden(p, y1, dinv, b1r)
    q = _sc_aggregate(src_pad, dst_pad, g, D_H)
    return _tc_final(q, g, dinv, W2, b2r)


# speedup vs baseline: 1.9522x; 1.0575x over previous
import functools

import jax
import jax.numpy as jnp
from jax import lax
from jax.experimental import pallas as pl
from jax.experimental.pallas import tpu as pltpu
from jax.experimental.pallas import tpu_sc as plsc

N = 10000
E = 320000
D_IN = 128
D_H = 128
D_OUT = 64

NC = 2
NS = 16
NW = NC * NS
EPW = E // NW
CH = 80
NCHUNK = EPW // CH
RPS = N // NS
DEGW = 128

@functools.cache
def _mesh():
    return plsc.VectorSubcoreMesh(
        core_axis_name="c", subcore_axis_name="s",
        num_cores=NC, num_subcores=NS)


def _sc_degree(dst):
    ones_rows = jnp.ones((CH, DEGW), jnp.float32)
    zero_rows = jnp.zeros((RPS, DEGW), jnp.float32)

    @functools.partial(
        pl.kernel,
        out_type=jax.ShapeDtypeStruct((NC, NS, RPS, DEGW), jnp.float32),
        mesh=_mesh(),
        scratch_types=[
            pltpu.VMEM((CH,), jnp.int32),
            pltpu.VMEM((CH, DEGW), jnp.float32),
            pltpu.VMEM_SHARED((N, DEGW), jnp.float32),
        ],
    )
    def k(dst_hbm, ones_hbm, zero_hbm, out_hbm, dstv, onesv, acc):
        c = lax.axis_index("c")
        s = lax.axis_index("s")
        pltpu.sync_copy(zero_hbm, acc.at[pl.ds(s * RPS, RPS)])
        pltpu.sync_copy(ones_hbm, onesv)
        plsc.subcore_barrier()
        base_w = (c * NS + s) * EPW

        @pl.loop(0, NCHUNK)
        def _(ci):
            pltpu.sync_copy(dst_hbm.at[pl.ds(base_w + ci * CH, CH)], dstv)
            pltpu.sync_copy(onesv, acc.at[dstv], add=True)

        plsc.subcore_barrier()
        pltpu.sync_copy(acc.at[pl.ds(s * RPS, RPS)], out_hbm.at[c, s])

    return k(dst, ones_rows, zero_rows).reshape(NC, N, DEGW)


CHP = 120
NCK0 = 100
NCK1 = 68
EPW0 = CHP * NCK0
EPW1 = CHP * NCK1
E_P = NS * (EPW0 + EPW1)
ACC_N = 10080
RPS_P = ACC_N // NS


def _sc_aggregate(src_pad, dst_pad, y, d):
    zero_rows = jnp.zeros((RPS_P, d), jnp.float32)

    @functools.partial(
        pl.kernel,
        out_type=jax.ShapeDtypeStruct((NC, NS, RPS_P, d), jnp.float32),
        mesh=_mesh(),
        scratch_types=[
            pltpu.VMEM((CHP,), jnp.int32),
            pltpu.VMEM((CHP,), jnp.int32),
            pltpu.VMEM((CHP, d), jnp.float32),
            pltpu.VMEM((CHP,), jnp.int32),
            pltpu.VMEM((CHP,), jnp.int32),
            pltpu.VMEM((CHP, d), jnp.float32),
            pltpu.VMEM_SHARED((ACC_N, d), jnp.float32),
            pltpu.SemaphoreType.DMA,
            pltpu.SemaphoreType.DMA,
        ],
    )
    def k(src_hbm, dst_hbm, y_hbm, zero_hbm, out_hbm,
          sa, da, ra, sb, db, rb, acc, gsa, gsb):
        c = lax.axis_index("c")
        s = lax.axis_index("s")
        pltpu.sync_copy(zero_hbm, acc.at[pl.ds(s * RPS_P, RPS_P)])
        plsc.subcore_barrier()
        base_w = jnp.where(c == 0, s * EPW0, NS * EPW0 + s * EPW1)
        nck = jnp.where(c == 0, NCK0, NCK1)

        def load_idx(ci, sbuf, dbuf):
            pltpu.sync_copy(src_hbm.at[pl.ds(base_w + ci * CHP, CHP)], sbuf)
            pltpu.sync_copy(dst_hbm.at[pl.ds(base_w + ci * CHP, CHP)], dbuf)

        load_idx(0, sa, da)
        pltpu.async_copy(y_hbm.at[sa], ra, gsa)
        load_idx(1, sb, db)
        pltpu.async_copy(y_hbm.at[sb], rb, gsb)

        @pl.loop(0, nck, step=2)
        def _(i):
            pltpu.make_async_copy(y_hbm.at[sa], ra, gsa).wait()
            pltpu.sync_copy(ra, acc.at[da], add=True)

            @pl.when(i + 2 < nck)
            def _():
                load_idx(i + 2, sa, da)
                pltpu.async_copy(y_hbm.at[sa], ra, gsa)

            pltpu.make_async_copy(y_hbm.at[sb], rb, gsb).wait()
            pltpu.sync_copy(rb, acc.at[db], add=True)

            @pl.when(i + 3 < nck)
            def _():
                load_idx(i + 3, sb, db)
                pltpu.async_copy(y_hbm.at[sb], rb, gsb)

        plsc.subcore_barrier()
        pltpu.sync_copy(acc.at[pl.ds(s * RPS_P, RPS_P)], out_hbm.at[c, s])

    return k(src_pad, dst_pad, y, zero_rows).reshape(NC, ACC_N, d)


_RB = 2000


def _tc_matmul(x, w):
    m, k = x.shape
    n = w.shape[1]

    def body(x_ref, w_ref, o_ref):
        o_ref[...] = jnp.dot(x_ref[...], w_ref[...],
                             preferred_element_type=jnp.float32)

    return pl.pallas_call(
        body,
        grid=(m // _RB,),
        in_specs=[pl.BlockSpec((_RB, k), lambda i: (i, 0)),
                  pl.BlockSpec((k, n), lambda i: (0, 0))],
        out_specs=pl.BlockSpec((_RB, n), lambda i: (i, 0)),
        out_shape=jax.ShapeDtypeStruct((m, n), jnp.float32),
    )(x, w)


def _tc_scale(xw, degp):
    def body(xw_ref, deg_ref, y_ref, dinv_ref):
        cnt = deg_ref[0, :, 0:1] + deg_ref[1, :, 0:1]
        dinv = lax.rsqrt(cnt + 1.0)
        dinv_ref[...] = dinv
        y_ref[...] = dinv * xw_ref[...]

    return pl.pallas_call(
        body,
        grid=(N // _RB,),
        in_specs=[pl.BlockSpec((_RB, D_H), lambda i: (i, 0)),
                  pl.BlockSpec((NC, _RB, DEGW), lambda i: (0, i, 0))],
        out_specs=[pl.BlockSpec((_RB, D_H), lambda i: (i, 0)),
                   pl.BlockSpec((_RB, 1), lambda i: (i, 0))],
        out_shape=[jax.ShapeDtypeStruct((N, D_H), jnp.float32),
                   jax.ShapeDtypeStruct((N, 1), jnp.float32)],
    )(xw, degp)


def _tc_hidden(p, y1, dinv, b1):
    def body(p_ref, y1_ref, dinv_ref, b1_ref, g_ref):
        agg = p_ref[0] + p_ref[1] + y1_ref[...]
        h = jnp.maximum(dinv_ref[...] * agg + b1_ref[...], 0.0)
        g_ref[...] = dinv_ref[...] * h

    return pl.pallas_call(
        body,
        grid=(N // _RB,),
        in_specs=[pl.BlockSpec((NC, _RB, D_H), lambda i: (0, i, 0)),
                  pl.BlockSpec((_RB, D_H), lambda i: (i, 0)),
                  pl.BlockSpec((_RB, 1), lambda i: (i, 0)),
                  pl.BlockSpec((1, D_H), lambda i: (0, 0))],
        out_specs=pl.BlockSpec((_RB, D_H), lambda i: (i, 0)),
        out_shape=jax.ShapeDtypeStruct((N, D_H), jnp.float32),
    )(p, y1, dinv, b1)


def _tc_final(q, g, dinv, w2, b2):
    def body(q_ref, g_ref, dinv_ref, w2_ref, b2_ref, o_ref):
        agg = dinv_ref[...] * (q_ref[0] + q_ref[1] + g_ref[...])
        o = jnp.dot(agg, w2_ref[...],
                    preferred_element_type=jnp.float32) + b2_ref[...]
        m = jnp.max(o, axis=1, keepdims=True)
        lse = jnp.log(jnp.sum(jnp.exp(o - m), axis=1, keepdims=True)) + m
        o_ref[...] = o - lse

    return pl.pallas_call(
        body,
        grid=(N // _RB,),
        in_specs=[pl.BlockSpec((NC, _RB, D_H), lambda i: (0, i, 0)),
                  pl.BlockSpec((_RB, D_H), lambda i: (i, 0)),
                  pl.BlockSpec((_RB, 1), lambda i: (i, 0)),
                  pl.BlockSpec((D_H, D_OUT), lambda i: (0, 0)),
                  pl.BlockSpec((1, D_OUT), lambda i: (0, 0))],
        out_specs=pl.BlockSpec((_RB, D_OUT), lambda i: (i, 0)),
        out_shape=jax.ShapeDtypeStruct((N, D_OUT), jnp.float32),
    )(q, g, dinv, w2, b2)


def kernel(x, edge_index, W1, b1, W2, b2):
    src = edge_index[0]
    dst = edge_index[1]
    pad = E_P - E
    src_pad = jnp.concatenate([src, jnp.zeros((pad,), jnp.int32)])
    fake_dst = N + (jnp.arange(pad, dtype=jnp.int32) % (ACC_N - N))
    dst_pad = jnp.concatenate([dst, fake_dst])
    b1r = b1.reshape(1, D_H)
    b2r = b2.reshape(1, D_OUT)

    degp = _sc_degree(dst)
    xw1 = _tc_matmul(x, W1)
    y1, dinv = _tc_scale(xw1, degp)
    p = _sc_aggregate(src_pad, dst_pad, y1, D_H)
    g = _tc_hidden(p, y1, dinv, b1r)
    q = _sc_aggregate(src_pad, dst_pad, g, D_H)
    return _tc_final(q, g, dinv, W2, b2r)


# speedup vs baseline: 1.9743x; 1.0113x over previous
import functools

import jax
import jax.numpy as jnp
from jax import lax
from jax.experimental import pallas as pl
from jax.experimental.pallas import tpu as pltpu
from jax.experimental.pallas import tpu_sc as plsc

N = 10000
E = 320000
D_IN = 128
D_H = 128
D_OUT = 64

NC = 2
NS = 16
NW = NC * NS
EPW = E // NW
CH = 80
NCHUNK = EPW // CH
RPS = N // NS
DEGW = 128

@functools.cache
def _mesh():
    return plsc.VectorSubcoreMesh(
        core_axis_name="c", subcore_axis_name="s",
        num_cores=NC, num_subcores=NS)


def _sc_degree(dst):
    ones_rows = jnp.ones((CH, DEGW), jnp.float32)
    zero_rows = jnp.zeros((RPS, DEGW), jnp.float32)

    @functools.partial(
        pl.kernel,
        out_type=jax.ShapeDtypeStruct((NC, NS, RPS, DEGW), jnp.float32),
        mesh=_mesh(),
        scratch_types=[
            pltpu.VMEM((CH,), jnp.int32),
            pltpu.VMEM((CH, DEGW), jnp.float32),
            pltpu.VMEM_SHARED((N, DEGW), jnp.float32),
        ],
    )
    def k(dst_hbm, ones_hbm, zero_hbm, out_hbm, dstv, onesv, acc):
        c = lax.axis_index("c")
        s = lax.axis_index("s")
        pltpu.sync_copy(zero_hbm, acc.at[pl.ds(s * RPS, RPS)])
        pltpu.sync_copy(ones_hbm, onesv)
        plsc.subcore_barrier()
        base_w = (c * NS + s) * EPW

        @pl.loop(0, NCHUNK)
        def _(ci):
            pltpu.sync_copy(dst_hbm.at[pl.ds(base_w + ci * CH, CH)], dstv)
            pltpu.sync_copy(onesv, acc.at[dstv], add=True)

        plsc.subcore_barrier()
        pltpu.sync_copy(acc.at[pl.ds(s * RPS, RPS)], out_hbm.at[c, s])

    return k(dst, ones_rows, zero_rows).reshape(NC, N, DEGW)


CHP = 120
NCK0 = 104
NCK1 = 64
EPW0 = CHP * NCK0
EPW1 = CHP * NCK1
E_P = NS * (EPW0 + EPW1)
ACC_N = 10080
RPS_P = ACC_N // NS


def _sc_aggregate(src_pad, dst_pad, y, d):
    zero_rows = jnp.zeros((RPS_P, d), jnp.float32)

    @functools.partial(
        pl.kernel,
        out_type=jax.ShapeDtypeStruct((NC, NS, RPS_P, d), jnp.float32),
        mesh=_mesh(),
        scratch_types=[
            pltpu.VMEM((CHP,), jnp.int32),
            pltpu.VMEM((CHP,), jnp.int32),
            pltpu.VMEM((CHP, d), jnp.float32),
            pltpu.VMEM((CHP,), jnp.int32),
            pltpu.VMEM((CHP,), jnp.int32),
            pltpu.VMEM((CHP, d), jnp.float32),
            pltpu.VMEM_SHARED((ACC_N, d), jnp.float32),
            pltpu.SemaphoreType.DMA,
            pltpu.SemaphoreType.DMA,
        ],
    )
    def k(src_hbm, dst_hbm, y_hbm, zero_hbm, out_hbm,
          sa, da, ra, sb, db, rb, acc, gsa, gsb):
        c = lax.axis_index("c")
        s = lax.axis_index("s")
        pltpu.sync_copy(zero_hbm, acc.at[pl.ds(s * RPS_P, RPS_P)])
        plsc.subcore_barrier()
        base_w = jnp.where(c == 0, s * EPW0, NS * EPW0 + s * EPW1)
        nck = jnp.where(c == 0, NCK0, NCK1)

        def load_idx(ci, sbuf, dbuf):
            pltpu.sync_copy(src_hbm.at[pl.ds(base_w + ci * CHP, CHP)], sbuf)
            pltpu.sync_copy(dst_hbm.at[pl.ds(base_w + ci * CHP, CHP)], dbuf)

        load_idx(0, sa, da)
        pltpu.async_copy(y_hbm.at[sa], ra, gsa)
        load_idx(1, sb, db)
        pltpu.async_copy(y_hbm.at[sb], rb, gsb)

        @pl.loop(0, nck, step=2)
        def _(i):
            pltpu.make_async_copy(y_hbm.at[sa], ra, gsa).wait()
            pltpu.sync_copy(ra, acc.at[da], add=True)

            @pl.when(i + 2 < nck)
            def _():
                load_idx(i + 2, sa, da)
                pltpu.async_copy(y_hbm.at[sa], ra, gsa)

            pltpu.make_async_copy(y_hbm.at[sb], rb, gsb).wait()
            pltpu.sync_copy(rb, acc.at[db], add=True)

            @pl.when(i + 3 < nck)
            def _():
                load_idx(i + 3, sb, db)
                pltpu.async_copy(y_hbm.at[sb], rb, gsb)

        plsc.subcore_barrier()
        pltpu.sync_copy(acc.at[pl.ds(s * RPS_P, RPS_P)], out_hbm.at[c, s])

    return k(src_pad, dst_pad, y, zero_rows).reshape(NC, ACC_N, d)


_RB = 2000


def _tc_matmul(x, w):
    m, k = x.shape
    n = w.shape[1]

    def body(x_ref, w_ref, o_ref):
        o_ref[...] = jnp.dot(x_ref[...], w_ref[...],
                             preferred_element_type=jnp.float32)

    return pl.pallas_call(
        body,
        grid=(m // _RB,),
        in_specs=[pl.BlockSpec((_RB, k), lambda i: (i, 0)),
                  pl.BlockSpec((k, n), lambda i: (0, 0))],
        out_specs=pl.BlockSpec((_RB, n), lambda i: (i, 0)),
        out_shape=jax.ShapeDtypeStruct((m, n), jnp.float32),
    )(x, w)


def _tc_scale(xw, degp):
    def body(xw_ref, deg_ref, y_ref, dinv_ref):
        cnt = deg_ref[0, :, 0:1] + deg_ref[1, :, 0:1]
        dinv = lax.rsqrt(cnt + 1.0)
        dinv_ref[...] = dinv
        y_ref[...] = dinv * xw_ref[...]

    return pl.pallas_call(
        body,
        grid=(N // _RB,),
        in_specs=[pl.BlockSpec((_RB, D_H), lambda i: (i, 0)),
                  pl.BlockSpec((NC, _RB, DEGW), lambda i: (0, i, 0))],
        out_specs=[pl.BlockSpec((_RB, D_H), lambda i: (i, 0)),
                   pl.BlockSpec((_RB, 1), lambda i: (i, 0))],
        out_shape=[jax.ShapeDtypeStruct((N, D_H), jnp.float32),
                   jax.ShapeDtypeStruct((N, 1), jnp.float32)],
    )(xw, degp)


def _tc_hidden(p, y1, dinv, b1):
    def body(p_ref, y1_ref, dinv_ref, b1_ref, g_ref):
        agg = p_ref[0] + p_ref[1] + y1_ref[...]
        h = jnp.maximum(dinv_ref[...] * agg + b1_ref[...], 0.0)
        g_ref[...] = dinv_ref[...] * h

    return pl.pallas_call(
        body,
        grid=(N // _RB,),
        in_specs=[pl.BlockSpec((NC, _RB, D_H), lambda i: (0, i, 0)),
                  pl.BlockSpec((_RB, D_H), lambda i: (i, 0)),
                  pl.BlockSpec((_RB, 1), lambda i: (i, 0)),
                  pl.BlockSpec((1, D_H), lambda i: (0, 0))],
        out_specs=pl.BlockSpec((_RB, D_H), lambda i: (i, 0)),
        out_shape=jax.ShapeDtypeStruct((N, D_H), jnp.float32),
    )(p, y1, dinv, b1)


def _tc_final(q, g, dinv, w2, b2):
    def body(q_ref, g_ref, dinv_ref, w2_ref, b2_ref, o_ref):
        agg = dinv_ref[...] * (q_ref[0] + q_ref[1] + g_ref[...])
        o = jnp.dot(agg, w2_ref[...],
                    preferred_element_type=jnp.float32) + b2_ref[...]
        m = jnp.max(o, axis=1, keepdims=True)
        lse = jnp.log(jnp.sum(jnp.exp(o - m), axis=1, keepdims=True)) + m
        o_ref[...] = o - lse

    return pl.pallas_call(
        body,
        grid=(N // _RB,),
        in_specs=[pl.BlockSpec((NC, _RB, D_H), lambda i: (0, i, 0)),
                  pl.BlockSpec((_RB, D_H), lambda i: (i, 0)),
                  pl.BlockSpec((_RB, 1), lambda i: (i, 0)),
                  pl.BlockSpec((D_H, D_OUT), lambda i: (0, 0)),
                  pl.BlockSpec((1, D_OUT), lambda i: (0, 0))],
        out_specs=pl.BlockSpec((_RB, D_OUT), lambda i: (i, 0)),
        out_shape=jax.ShapeDtypeStruct((N, D_OUT), jnp.float32),
    )(q, g, dinv, w2, b2)


def kernel(x, edge_index, W1, b1, W2, b2):
    src = edge_index[0]
    dst = edge_index[1]
    pad = E_P - E
    src_pad = jnp.concatenate([src, jnp.zeros((pad,), jnp.int32)])
    fake_dst = N + (jnp.arange(pad, dtype=jnp.int32) % (ACC_N - N))
    dst_pad = jnp.concatenate([dst, fake_dst])
    b1r = b1.reshape(1, D_H)
    b2r = b2.reshape(1, D_OUT)

    degp = _sc_degree(dst)
    xw1 = _tc_matmul(x, W1)
    y1, dinv = _tc_scale(xw1, degp)
    p = _sc_aggregate(src_pad, dst_pad, y1, D_H)
    g = _tc_hidden(p, y1, dinv, b1r)
    q = _sc_aggregate(src_pad, dst_pad, g, D_H)
    return _tc_final(q, g, dinv, W2, b2r)


# speedup vs baseline: 1.9998x; 1.0129x over previous
import functools

import jax
import jax.numpy as jnp
from jax import lax
from jax.experimental import pallas as pl
from jax.experimental.pallas import tpu as pltpu
from jax.experimental.pallas import tpu_sc as plsc

N = 10000
E = 320000
D_IN = 128
D_H = 128
D_OUT = 64

NC = 2
NS = 16
NW = NC * NS
EPW = E // NW
CH = 80
NCHUNK = EPW // CH
RPS = N // NS
DEGW = 128

@functools.cache
def _mesh():
    return plsc.VectorSubcoreMesh(
        core_axis_name="c", subcore_axis_name="s",
        num_cores=NC, num_subcores=NS)


def _sc_degree(dst):
    ones_rows = jnp.ones((CH, DEGW), jnp.float32)
    zero_rows = jnp.zeros((RPS, DEGW), jnp.float32)

    @functools.partial(
        pl.kernel,
        out_type=jax.ShapeDtypeStruct((NC, NS, RPS, DEGW), jnp.float32),
        mesh=_mesh(),
        scratch_types=[
            pltpu.VMEM((CH,), jnp.int32),
            pltpu.VMEM((CH, DEGW), jnp.float32),
            pltpu.VMEM_SHARED((N, DEGW), jnp.float32),
        ],
    )
    def k(dst_hbm, ones_hbm, zero_hbm, out_hbm, dstv, onesv, acc):
        c = lax.axis_index("c")
        s = lax.axis_index("s")
        pltpu.sync_copy(zero_hbm, acc.at[pl.ds(s * RPS, RPS)])
        pltpu.sync_copy(ones_hbm, onesv)
        plsc.subcore_barrier()
        base_w = (c * NS + s) * EPW

        @pl.loop(0, NCHUNK)
        def _(ci):
            pltpu.sync_copy(dst_hbm.at[pl.ds(base_w + ci * CH, CH)], dstv)
            pltpu.sync_copy(onesv, acc.at[dstv], add=True)

        plsc.subcore_barrier()
        pltpu.sync_copy(acc.at[pl.ds(s * RPS, RPS)], out_hbm.at[c, s])

    return k(dst, ones_rows, zero_rows).reshape(NC, N, DEGW)


CHP = 120
NCK0 = 108
NCK1 = 60
EPW0 = CHP * NCK0
EPW1 = CHP * NCK1
E_P = NS * (EPW0 + EPW1)
ACC_N = 10080
RPS_P = ACC_N // NS


def _sc_aggregate(src_pad, dst_pad, y, d):
    zero_rows = jnp.zeros((RPS_P, d), jnp.float32)

    @functools.partial(
        pl.kernel,
        out_type=jax.ShapeDtypeStruct((NC, NS, RPS_P, d), jnp.float32),
        mesh=_mesh(),
        scratch_types=[
            pltpu.VMEM((CHP,), jnp.int32),
            pltpu.VMEM((CHP,), jnp.int32),
            pltpu.VMEM((CHP, d), jnp.float32),
            pltpu.VMEM((CHP,), jnp.int32),
            pltpu.VMEM((CHP,), jnp.int32),
            pltpu.VMEM((CHP, d), jnp.float32),
            pltpu.VMEM_SHARED((ACC_N, d), jnp.float32),
            pltpu.SemaphoreType.DMA,
            pltpu.SemaphoreType.DMA,
        ],
    )
    def k(src_hbm, dst_hbm, y_hbm, zero_hbm, out_hbm,
          sa, da, ra, sb, db, rb, acc, gsa, gsb):
        c = lax.axis_index("c")
        s = lax.axis_index("s")
        pltpu.sync_copy(zero_hbm, acc.at[pl.ds(s * RPS_P, RPS_P)])
        plsc.subcore_barrier()
        base_w = jnp.where(c == 0, s * EPW0, NS * EPW0 + s * EPW1)
        nck = jnp.where(c == 0, NCK0, NCK1)

        def load_idx(ci, sbuf, dbuf):
            pltpu.sync_copy(src_hbm.at[pl.ds(base_w + ci * CHP, CHP)], sbuf)
            pltpu.sync_copy(dst_hbm.at[pl.ds(base_w + ci * CHP, CHP)], dbuf)

        load_idx(0, sa, da)
        pltpu.async_copy(y_hbm.at[sa], ra, gsa)
        load_idx(1, sb, db)
        pltpu.async_copy(y_hbm.at[sb], rb, gsb)

        @pl.loop(0, nck, step=2)
        def _(i):
            pltpu.make_async_copy(y_hbm.at[sa], ra, gsa).wait()
            pltpu.sync_copy(ra, acc.at[da], add=True)

            @pl.when(i + 2 < nck)
            def _():
                load_idx(i + 2, sa, da)
                pltpu.async_copy(y_hbm.at[sa], ra, gsa)

            pltpu.make_async_copy(y_hbm.at[sb], rb, gsb).wait()
            pltpu.sync_copy(rb, acc.at[db], add=True)

            @pl.when(i + 3 < nck)
            def _():
                load_idx(i + 3, sb, db)
                pltpu.async_copy(y_hbm.at[sb], rb, gsb)

        plsc.subcore_barrier()
        pltpu.sync_copy(acc.at[pl.ds(s * RPS_P, RPS_P)], out_hbm.at[c, s])

    return k(src_pad, dst_pad, y, zero_rows).reshape(NC, ACC_N, d)


_RB = 2000


def _tc_matmul(x, w):
    m, k = x.shape
    n = w.shape[1]

    def body(x_ref, w_ref, o_ref):
        o_ref[...] = jnp.dot(x_ref[...], w_ref[...],
                             preferred_element_type=jnp.float32)

    return pl.pallas_call(
        body,
        grid=(m // _RB,),
        in_specs=[pl.BlockSpec((_RB, k), lambda i: (i, 0)),
                  pl.BlockSpec((k, n), lambda i: (0, 0))],
        out_specs=pl.BlockSpec((_RB, n), lambda i: (i, 0)),
        out_shape=jax.ShapeDtypeStruct((m, n), jnp.float32),
    )(x, w)


def _tc_scale(xw, degp):
    def body(xw_ref, deg_ref, y_ref, dinv_ref):
        cnt = deg_ref[0, :, 0:1] + deg_ref[1, :, 0:1]
        dinv = lax.rsqrt(cnt + 1.0)
        dinv_ref[...] = dinv
        y_ref[...] = dinv * xw_ref[...]

    return pl.pallas_call(
        body,
        grid=(N // _RB,),
        in_specs=[pl.BlockSpec((_RB, D_H), lambda i: (i, 0)),
                  pl.BlockSpec((NC, _RB, DEGW), lambda i: (0, i, 0))],
        out_specs=[pl.BlockSpec((_RB, D_H), lambda i: (i, 0)),
                   pl.BlockSpec((_RB, 1), lambda i: (i, 0))],
        out_shape=[jax.ShapeDtypeStruct((N, D_H), jnp.float32),
                   jax.ShapeDtypeStruct((N, 1), jnp.float32)],
    )(xw, degp)


def _tc_hidden(p, y1, dinv, b1):
    def body(p_ref, y1_ref, dinv_ref, b1_ref, g_ref):
        agg = p_ref[0] + p_ref[1] + y1_ref[...]
        h = jnp.maximum(dinv_ref[...] * agg + b1_ref[...], 0.0)
        g_ref[...] = dinv_ref[...] * h

    return pl.pallas_call(
        body,
        grid=(N // _RB,),
        in_specs=[pl.BlockSpec((NC, _RB, D_H), lambda i: (0, i, 0)),
                  pl.BlockSpec((_RB, D_H), lambda i: (i, 0)),
                  pl.BlockSpec((_RB, 1), lambda i: (i, 0)),
                  pl.BlockSpec((1, D_H), lambda i: (0, 0))],
        out_specs=pl.BlockSpec((_RB, D_H), lambda i: (i, 0)),
        out_shape=jax.ShapeDtypeStruct((N, D_H), jnp.float32),
    )(p, y1, dinv, b1)


def _tc_final(q, g, dinv, w2, b2):
    def body(q_ref, g_ref, dinv_ref, w2_ref, b2_ref, o_ref):
        agg = dinv_ref[...] * (q_ref[0] + q_ref[1] + g_ref[...])
        o = jnp.dot(agg, w2_ref[...],
                    preferred_element_type=jnp.float32) + b2_ref[...]
        m = jnp.max(o, axis=1, keepdims=True)
        lse = jnp.log(jnp.sum(jnp.exp(o - m), axis=1, keepdims=True)) + m
        o_ref[...] = o - lse

    return pl.pallas_call(
        body,
        grid=(N // _RB,),
        in_specs=[pl.BlockSpec((NC, _RB, D_H), lambda i: (0, i, 0)),
                  pl.BlockSpec((_RB, D_H), lambda i: (i, 0)),
                  pl.BlockSpec((_RB, 1), lambda i: (i, 0)),
                  pl.BlockSpec((D_H, D_OUT), lambda i: (0, 0)),
                  pl.BlockSpec((1, D_OUT), lambda i: (0, 0))],
        out_specs=pl.BlockSpec((_RB, D_OUT), lambda i: (i, 0)),
        out_shape=jax.ShapeDtypeStruct((N, D_OUT), jnp.float32),
    )(q, g, dinv, w2, b2)


def kernel(x, edge_index, W1, b1, W2, b2):
    src = edge_index[0]
    dst = edge_index[1]
    pad = E_P - E
    src_pad = jnp.concatenate([src, jnp.zeros((pad,), jnp.int32)])
    fake_dst = N + (jnp.arange(pad, dtype=jnp.int32) % (ACC_N - N))
    dst_pad = jnp.concatenate([dst, fake_dst])
    b1r = b1.reshape(1, D_H)
    b2r = b2.reshape(1, D_OUT)

    degp = _sc_degree(dst)
    xw1 = _tc_matmul(x, W1)
    y1, dinv = _tc_scale(xw1, degp)
    p = _sc_aggregate(src_pad, dst_pad, y1, D_H)
    g = _tc_hidden(p, y1, dinv, b1r)
    q = _sc_aggregate(src_pad, dst_pad, g, D_H)
    return _tc_final(q, g, dinv, W2, b2r)


# speedup vs baseline: 2.0332x; 1.0167x over previous
import functools

import jax
import jax.numpy as jnp
from jax import lax
from jax.experimental import pallas as pl
from jax.experimental.pallas import tpu as pltpu
from jax.experimental.pallas import tpu_sc as plsc

N = 10000
E = 320000
D_IN = 128
D_H = 128
D_OUT = 64

NC = 2
NS = 16
NW = NC * NS
EPW = E // NW
CH = 80
NCHUNK = EPW // CH
RPS = N // NS
DEGW = 128

@functools.cache
def _mesh():
    return plsc.VectorSubcoreMesh(
        core_axis_name="c", subcore_axis_name="s",
        num_cores=NC, num_subcores=NS)


def _sc_degree(dst):
    ones_rows = jnp.ones((CH, DEGW), jnp.float32)
    zero_rows = jnp.zeros((RPS, DEGW), jnp.float32)

    @functools.partial(
        pl.kernel,
        out_type=jax.ShapeDtypeStruct((NC, NS, RPS, DEGW), jnp.float32),
        mesh=_mesh(),
        scratch_types=[
            pltpu.VMEM((CH,), jnp.int32),
            pltpu.VMEM((CH, DEGW), jnp.float32),
            pltpu.VMEM_SHARED((N, DEGW), jnp.float32),
        ],
    )
    def k(dst_hbm, ones_hbm, zero_hbm, out_hbm, dstv, onesv, acc):
        c = lax.axis_index("c")
        s = lax.axis_index("s")
        pltpu.sync_copy(zero_hbm, acc.at[pl.ds(s * RPS, RPS)])
        pltpu.sync_copy(ones_hbm, onesv)
        plsc.subcore_barrier()
        base_w = (c * NS + s) * EPW

        @pl.loop(0, NCHUNK)
        def _(ci):
            pltpu.sync_copy(dst_hbm.at[pl.ds(base_w + ci * CH, CH)], dstv)
            pltpu.sync_copy(onesv, acc.at[dstv], add=True)

        plsc.subcore_barrier()
        pltpu.sync_copy(acc.at[pl.ds(s * RPS, RPS)], out_hbm.at[c, s])

    return k(dst, ones_rows, zero_rows).reshape(NC, N, DEGW)


CHP = 120
NCK0 = 112
NCK1 = 56
EPW0 = CHP * NCK0
EPW1 = CHP * NCK1
E_P = NS * (EPW0 + EPW1)
ACC_N = 10080
RPS_P = ACC_N // NS


def _sc_aggregate(src_pad, dst_pad, y, d):
    zero_rows = jnp.zeros((RPS_P, d), jnp.float32)

    @functools.partial(
        pl.kernel,
        out_type=jax.ShapeDtypeStruct((NC, NS, RPS_P, d), jnp.float32),
        mesh=_mesh(),
        scratch_types=[
            pltpu.VMEM((CHP,), jnp.int32),
            pltpu.VMEM((CHP,), jnp.int32),
            pltpu.VMEM((CHP, d), jnp.float32),
            pltpu.VMEM((CHP,), jnp.int32),
            pltpu.VMEM((CHP,), jnp.int32),
            pltpu.VMEM((CHP, d), jnp.float32),
            pltpu.VMEM_SHARED((ACC_N, d), jnp.float32),
            pltpu.SemaphoreType.DMA,
            pltpu.SemaphoreType.DMA,
        ],
    )
    def k(src_hbm, dst_hbm, y_hbm, zero_hbm, out_hbm,
          sa, da, ra, sb, db, rb, acc, gsa, gsb):
        c = lax.axis_index("c")
        s = lax.axis_index("s")
        pltpu.sync_copy(zero_hbm, acc.at[pl.ds(s * RPS_P, RPS_P)])
        plsc.subcore_barrier()
        base_w = jnp.where(c == 0, s * EPW0, NS * EPW0 + s * EPW1)
        nck = jnp.where(c == 0, NCK0, NCK1)

        def load_idx(ci, sbuf, dbuf):
            pltpu.sync_copy(src_hbm.at[pl.ds(base_w + ci * CHP, CHP)], sbuf)
            pltpu.sync_copy(dst_hbm.at[pl.ds(base_w + ci * CHP, CHP)], dbuf)

        load_idx(0, sa, da)
        pltpu.async_copy(y_hbm.at[sa], ra, gsa)
        load_idx(1, sb, db)
        pltpu.async_copy(y_hbm.at[sb], rb, gsb)

        @pl.loop(0, nck, step=2)
        def _(i):
            pltpu.make_async_copy(y_hbm.at[sa], ra, gsa).wait()
            pltpu.sync_copy(ra, acc.at[da], add=True)

            @pl.when(i + 2 < nck)
            def _():
                load_idx(i + 2, sa, da)
                pltpu.async_copy(y_hbm.at[sa], ra, gsa)

            pltpu.make_async_copy(y_hbm.at[sb], rb, gsb).wait()
            pltpu.sync_copy(rb, acc.at[db], add=True)

            @pl.when(i + 3 < nck)
            def _():
                load_idx(i + 3, sb, db)
                pltpu.async_copy(y_hbm.at[sb], rb, gsb)

        plsc.subcore_barrier()
        pltpu.sync_copy(acc.at[pl.ds(s * RPS_P, RPS_P)], out_hbm.at[c, s])

    return k(src_pad, dst_pad, y, zero_rows).reshape(NC, ACC_N, d)


_RB = 2000


def _tc_matmul(x, w):
    m, k = x.shape
    n = w.shape[1]

    def body(x_ref, w_ref, o_ref):
        o_ref[...] = jnp.dot(x_ref[...], w_ref[...],
                             preferred_element_type=jnp.float32)

    return pl.pallas_call(
        body,
        grid=(m // _RB,),
        in_specs=[pl.BlockSpec((_RB, k), lambda i: (i, 0)),
                  pl.BlockSpec((k, n), lambda i: (0, 0))],
        out_specs=pl.BlockSpec((_RB, n), lambda i: (i, 0)),
        out_shape=jax.ShapeDtypeStruct((m, n), jnp.float32),
    )(x, w)


def _tc_scale(xw, degp):
    def body(xw_ref, deg_ref, y_ref, dinv_ref):
        cnt = deg_ref[0, :, 0:1] + deg_ref[1, :, 0:1]
        dinv = lax.rsqrt(cnt + 1.0)
        dinv_ref[...] = dinv
        y_ref[...] = dinv * xw_ref[...]

    return pl.pallas_call(
        body,
        grid=(N // _RB,),
        in_specs=[pl.BlockSpec((_RB, D_H), lambda i: (i, 0)),
                  pl.BlockSpec((NC, _RB, DEGW), lambda i: (0, i, 0))],
        out_specs=[pl.BlockSpec((_RB, D_H), lambda i: (i, 0)),
                   pl.BlockSpec((_RB, 1), lambda i: (i, 0))],
        out_shape=[jax.ShapeDtypeStruct((N, D_H), jnp.float32),
                   jax.ShapeDtypeStruct((N, 1), jnp.float32)],
    )(xw, degp)


def _tc_hidden(p, y1, dinv, b1):
    def body(p_ref, y1_ref, dinv_ref, b1_ref, g_ref):
        agg = p_ref[0] + p_ref[1] + y1_ref[...]
        h = jnp.maximum(dinv_ref[...] * agg + b1_ref[...], 0.0)
        g_ref[...] = dinv_ref[...] * h

    return pl.pallas_call(
        body,
        grid=(N // _RB,),
        in_specs=[pl.BlockSpec((NC, _RB, D_H), lambda i: (0, i, 0)),
                  pl.BlockSpec((_RB, D_H), lambda i: (i, 0)),
                  pl.BlockSpec((_RB, 1), lambda i: (i, 0)),
                  pl.BlockSpec((1, D_H), lambda i: (0, 0))],
        out_specs=pl.BlockSpec((_RB, D_H), lambda i: (i, 0)),
        out_shape=jax.ShapeDtypeStruct((N, D_H), jnp.float32),
    )(p, y1, dinv, b1)


def _tc_final(q, g, dinv, w2, b2):
    def body(q_ref, g_ref, dinv_ref, w2_ref, b2_ref, o_ref):
        agg = dinv_ref[...] * (q_ref[0] + q_ref[1] + g_ref[...])
        o = jnp.dot(agg, w2_ref[...],
                    preferred_element_type=jnp.float32) + b2_ref[...]
        m = jnp.max(o, axis=1, keepdims=True)
        lse = jnp.log(jnp.sum(jnp.exp(o - m), axis=1, keepdims=True)) + m
        o_ref[...] = o - lse

    return pl.pallas_call(
        body,
        grid=(N // _RB,),
        in_specs=[pl.BlockSpec((NC, _RB, D_H), lambda i: (0, i, 0)),
                  pl.BlockSpec((_RB, D_H), lambda i: (i, 0)),
                  pl.BlockSpec((_RB, 1), lambda i: (i, 0)),
                  pl.BlockSpec((D_H, D_OUT), lambda i: (0, 0)),
                  pl.BlockSpec((1, D_OUT), lambda i: (0, 0))],
        out_specs=pl.BlockSpec((_RB, D_OUT), lambda i: (i, 0)),
        out_shape=jax.ShapeDtypeStruct((N, D_OUT), jnp.float32),
    )(q, g, dinv, w2, b2)


def kernel(x, edge_index, W1, b1, W2, b2):
    src = edge_index[0]
    dst = edge_index[1]
    pad = E_P - E
    src_pad = jnp.concatenate([src, jnp.zeros((pad,), jnp.int32)])
    fake_dst = N + (jnp.arange(pad, dtype=jnp.int32) % (ACC_N - N))
    dst_pad = jnp.concatenate([dst, fake_dst])
    b1r = b1.reshape(1, D_H)
    b2r = b2.reshape(1, D_OUT)

    degp = _sc_degree(dst)
    xw1 = _tc_matmul(x, W1)
    y1, dinv = _tc_scale(xw1, degp)
    p = _sc_aggregate(src_pad, dst_pad, y1, D_H)
    g = _tc_hidden(p, y1, dinv, b1r)
    q = _sc_aggregate(src_pad, dst_pad, g, D_H)
    return _tc_final(q, g, dinv, W2, b2r)
